# baseline (speedup 1.0000x reference)
import functools

import jax
import jax.numpy as jnp
from jax import lax
from jax.experimental import pallas as pl
from jax.experimental.pallas import tpu as pltpu
from jax.experimental.pallas import tpu_sc as plsc

_WINDOW = 5
_NEG = 4

_NC = 2
_NS = 16
_NW = _NC * _NS


_CH = 640


def _sc_gather(table, idx_all):
    n = idx_all.shape[0]
    d = table.shape[1]
    per_w = n // _NW
    assert per_w % _CH == 0 and n % _NW == 0
    nch = per_w // _CH
    npair = (nch + 1) // 2

    mesh = plsc.VectorSubcoreMesh(
        core_axis_name="c", subcore_axis_name="s",
        num_cores=_NC, num_subcores=_NS)

    @functools.partial(
        pl.kernel,
        out_type=jax.ShapeDtypeStruct((n, d), jnp.float32),
        mesh=mesh,
        scratch_types=[
            pltpu.VMEM((per_w,), jnp.int32),
            pltpu.VMEM((2, _CH, d), jnp.float32),
            pltpu.SemaphoreType.DMA,
            pltpu.SemaphoreType.DMA,
        ],
        compiler_params=pltpu.CompilerParams(use_tc_tiling_on_sc=False),
    )
    def gather_kernel(table_hbm, idx_hbm, out_hbm, idx_v, rows_v, sem0, sem1):
        wid = lax.axis_index("s") * _NC + lax.axis_index("c")
        base = pl.multiple_of(wid * per_w, 8)
        pltpu.sync_copy(idx_hbm.at[pl.ds(base, per_w)], idx_v)
        sems = (sem0, sem1)

        def gather_chunk(i, b):
            off = pl.multiple_of(i * _CH, 8)
            return pltpu.make_async_copy(
                table_hbm.at[idx_v.at[pl.ds(off, _CH)]], rows_v.at[b], sems[b])

        gather_chunk(0, 0).start()

        def pair(j, carry):
            for b in range(2):
                i = 2 * j + b

                @pl.when(i + 1 < nch)
                def _():
                    gather_chunk(i + 1, 1 - b).start()

                @pl.when(i < nch)
                def _():
                    gather_chunk(i, b).wait()
                    pltpu.sync_copy(
                        rows_v.at[b],
                        out_hbm.at[pl.ds(pl.multiple_of(base + i * _CH, 8),
                                         _CH)])
            return carry

        lax.fori_loop(0, npair, pair, 0)

    return gather_kernel(table, idx_all)


def _rowsum(prod, ones_row):
    return lax.dot_general(ones_row, prod, (((1,), (1,)), ((), ())),
                           preferred_element_type=jnp.float32)


def _loss_body(w_ref, n0_ref, n1_ref, n2_ref, n3_ref, out_ref, *,
               t, r, nb, scale):
    pid = pl.program_id(0)
    d = w_ref.shape[-1]
    l = w_ref.shape[0] // r
    w = w_ref[...].reshape(r, l, d)
    a2 = w[:, :t, :].reshape(r * t, d)
    ones_row = jnp.ones((1, d), jnp.float32)
    acc = jnp.float32(0.0)
    for k in range(1, _WINDOW):
        p2 = w[:, k:t + k, :].reshape(r * t, d)
        s = _rowsum(a2 * p2, ones_row)
        acc += jnp.sum(jnp.maximum(-s, 0.0) + jnp.log1p(jnp.exp(-jnp.abs(s))))
    for n_ref in (n0_ref, n1_ref, n2_ref, n3_ref):
        n2 = n_ref[...]
        nl = _rowsum(a2 * n2, ones_row)
        acc += jnp.sum(jnp.maximum(nl, 0.0) + jnp.log1p(jnp.exp(-jnp.abs(nl))))

    @pl.when(pid == 0)
    def _():
        out_ref[...] = jnp.zeros_like(out_ref)

    out_ref[...] += acc.reshape(1, 1)

    @pl.when(pid == nb - 1)
    def _():
        out_ref[...] *= jnp.float32(scale)


def _tc_loss(gathered, b, l, t, walk_blocks, neg_block0):
    d = gathered.shape[-1]
    r = 64
    nb = b // r
    rt = r * t
    n_terms = b * t * (_WINDOW - 1 + _NEG)
    body = functools.partial(_loss_body, t=t, r=r, nb=nb,
                             scale=1.0 / float(n_terms))

    def neg_spec(k):
        return pl.BlockSpec((rt, d), lambda i, k=k: (neg_block0 + k * nb + i, 0))

    out = pl.pallas_call(
        body,
        grid=(nb,),
        in_specs=[pl.BlockSpec((r * l, d), lambda i: (i, 0))]
        + [neg_spec(k) for k in range(_NEG)],
        out_specs=pl.BlockSpec((1, 1), lambda i: (0, 0)),
        out_shape=jax.ShapeDtypeStruct((1, 1), jnp.float32),
    )(gathered, gathered, gathered, gathered, gathered)
    return out[0, 0]


def kernel(walk, table):
    b, l = walk.shape
    t = l - _WINDOW + 1
    bt = b * t
    n_nodes, d = table.shape
    neg = jax.random.randint(jax.random.key(42), (bt, _NEG), 1, n_nodes - 1,
                             dtype=jnp.int32)
    r = 64
    rt = r * t
    n_walk = b * l
    w_region = ((n_walk + rt - 1) // rt) * rt
    grain = _NW * _CH
    total = ((w_region + _NEG * bt + grain - 1) // grain) * grain
    pad1 = w_region - n_walk
    pad2 = total - w_region - _NEG * bt
    idx_all = jnp.concatenate([
        walk.reshape(-1),
        jnp.zeros((pad1,), jnp.int32),
        neg.T.reshape(-1),
        jnp.zeros((pad2,), jnp.int32),
    ])
    gathered = _sc_gather(table, idx_all)
    return _tc_loss(gathered, b, l, t, n_walk // (r * l), w_region // rt)
